# baseline (speedup 1.0000x reference)
import functools

import jax
import jax.numpy as jnp
from jax import lax
from jax.experimental import pallas as pl
from jax.experimental.pallas import tpu as pltpu
from jax.experimental.pallas import tpu_sc as plsc

N_NODES = 10000
N_EDGES = 320000
D = 128

NW = 32
EPW = N_EDGES // NW
C = 80
K = EPW // C
HALF = 5120
AGG_R = 5248
DUMP = HALF
RPE = HALF // 16
RPZ = AGG_R // 16
N_PAD = 10240


def _node_proj_body(x_ref, w_ref, b_ref, o_ref):
    o_ref[...] = jnp.dot(x_ref[...], w_ref[...],
                         preferred_element_type=jnp.float32) + b_ref[...]


def _node_proj(x, w, b):
    br = 2000
    return pl.pallas_call(
        _node_proj_body,
        grid=(N_NODES // br,),
        in_specs=[
            pl.BlockSpec((br, D), lambda i: (i, 0)),
            pl.BlockSpec((D, D), lambda i: (0, 0)),
            pl.BlockSpec((1, D), lambda i: (0, 0)),
        ],
        out_specs=pl.BlockSpec((br, D), lambda i: (i, 0)),
        out_shape=jax.ShapeDtypeStruct((N_NODES, D), jnp.float32),
    )(x, w, b)


def _update_body(x_ref, p_ref, w1_ref, w2_ref, b_ref, o_ref):
    agg = p_ref[0] + p_ref[1]
    o_ref[...] = jnp.tanh(
        jnp.dot(x_ref[...], w1_ref[...], preferred_element_type=jnp.float32)
        + jnp.dot(agg, w2_ref[...], preferred_element_type=jnp.float32)
        + b_ref[...])


def _update(x, parts, w1, w2, b):
    br = 2000
    return pl.pallas_call(
        _update_body,
        grid=(N_NODES // br,),
        in_specs=[
            pl.BlockSpec((br, D), lambda i: (i, 0)),
            pl.BlockSpec((2, br, D), lambda i: (0, i, 0)),
            pl.BlockSpec((D, D), lambda i: (0, 0)),
            pl.BlockSpec((D, D), lambda i: (0, 0)),
            pl.BlockSpec((1, D), lambda i: (0, 0)),
        ],
        out_specs=pl.BlockSpec((br, D), lambda i: (i, 0)),
        out_shape=jax.ShapeDtypeStruct((N_NODES, D), jnp.float32),
    )(x, parts, w1, w2, b)


@functools.partial(
    pl.kernel,
    out_type=[
        jax.ShapeDtypeStruct((2, N_PAD, D), jnp.float32),
        jax.ShapeDtypeStruct((N_EDGES, D), jnp.float32),
    ],
    mesh=plsc.VectorSubcoreMesh(core_axis_name="c", subcore_axis_name="s"),
    scratch_types=[
        pltpu.VMEM((EPW,), jnp.int32),
        pltpu.VMEM((EPW,), jnp.int32),
        pltpu.VMEM((EPW * 4,), jnp.float32),
        pltpu.VMEM((4, D), jnp.float32),
        pltpu.VMEM((2, C), jnp.int32),
        pltpu.VMEM((2, C, D), jnp.float32),
        pltpu.VMEM_SHARED((AGG_R, D), jnp.float32),
    ] + [pltpu.SemaphoreType.DMA] * 6,
)
def _sc_edge(p_hbm, ea_hbm, wme_hbm, src_hbm, dst_hbm, zeros_hbm,
             out_hbm, msg_hbm,
             src_sl, dst_sl, ea_sl, wme_v, idxt_b, rows_b, agg_sh,
             sg0, sg1, sp0, sp1, sc0, sc1):
    s_g = (sg0, sg1)
    s_sp = (sp0, sp1)
    s_sc = (sc0, sc1)

    cid = lax.axis_index("c")
    sid = lax.axis_index("s")
    wid = sid * 2 + cid
    base0 = wid * EPW
    zstripe = pl.ds(sid * RPZ, RPZ)
    estripe = pl.ds(sid * RPE, RPE)

    def esl(j):
        return pl.ds(base0 + j * C, C)

    def i_gather(j, b):
        pltpu.async_copy(p_hbm.at[src_sl.at[pl.ds(j * C, C)]],
                         rows_b.at[b], s_g[b])

    def w_gather(j, b):
        pltpu.make_async_copy(p_hbm.at[src_sl.at[pl.ds(j * C, C)]],
                              rows_b.at[b], s_g[b]).wait()

    def i_msg_rd(j, b):
        pltpu.async_copy(msg_hbm.at[esl(j)], rows_b.at[b], s_g[b])

    def w_msg_rd(j, b):
        pltpu.make_async_copy(msg_hbm.at[esl(j)], rows_b.at[b], s_g[b]).wait()

    def i_spill(j, b):
        pltpu.async_copy(rows_b.at[b], msg_hbm.at[esl(j)], s_sp[b])

    def w_spill(j, b):
        pltpu.make_async_copy(rows_b.at[b], msg_hbm.at[esl(j)],
                              s_sp[b]).wait()

    def i_scat(b):
        pltpu.async_copy(rows_b.at[b], agg_sh.at[idxt_b.at[b]], s_sc[b],
                         add=True)

    def w_scat(b):
        pltpu.make_async_copy(rows_b.at[b], agg_sh.at[idxt_b.at[b]],
                              s_sc[b]).wait()

    def compute(j, b):
        def egrp(q, c2):
            ev = ea_sl[pl.ds(j * (C * 4) + q * 16, 16)]
            for r in range(4):
                i = q * 4 + r
                e0 = ev[r * 4 + 0]
                e1 = ev[r * 4 + 1]
                e2 = ev[r * 4 + 2]
                e3 = ev[r * 4 + 3]
                for k8 in range(D // 16):
                    sl = pl.ds(k8 * 16, 16)
                    g = (rows_b[b, i, sl]
                         + e0 * wme_v[0, sl] + e1 * wme_v[1, sl]
                         + e2 * wme_v[2, sl] + e3 * wme_v[3, sl])
                    t = jnp.exp(-2.0 * jnp.abs(g))
                    rows_b[b, i, sl] = jnp.sign(g) * ((1.0 - t) / (1.0 + t))
            return c2

        lax.fori_loop(0, C // 4, egrp, 0)

    def clamp(j, b, lo):
        for k in range(C // 16):
            v = dst_sl[pl.ds(j * C + k * 16, 16)] - lo
            ok = jnp.logical_and(v >= 0, v < HALF)
            dump = DUMP + k * 16 + lax.broadcasted_iota(jnp.int32, (16,), 0)
            idxt_b[b, pl.ds(k * 16, 16)] = jnp.where(ok, v, dump)

    pltpu.sync_copy(src_hbm.at[pl.ds(base0, EPW)], src_sl)
    pltpu.sync_copy(dst_hbm.at[pl.ds(base0, EPW)], dst_sl)
    pltpu.sync_copy(ea_hbm.at[pl.ds(base0 * 4, EPW * 4)], ea_sl)
    pltpu.sync_copy(wme_hbm, wme_v)
    pltpu.sync_copy(zeros_hbm, agg_sh.at[zstripe])
    plsc.subcore_barrier()


    def a_step(j, b, nb, first, do_next_gather):
        if do_next_gather:
            if not first:
                w_spill(j - 1, nb)
                w_scat(nb)
            i_gather(j + 1, nb)
        w_gather(j, b)
        compute(j, b)
        clamp(j, b, 0)
        i_spill(j, b)
        i_scat(b)

    i_gather(0, 0)
    a_step(0, 0, 1, True, True)

    def a_pair(jj, carry):
        j = 1 + 2 * jj
        a_step(j, 1, 0, False, True)
        a_step(j + 1, 0, 1, False, True)
        return carry

    lax.fori_loop(0, 61, a_pair, 0)
    a_step(123, 1, 0, False, True)
    a_step(124, 0, 1, False, False)
    w_spill(123, 1)
    w_spill(124, 0)
    w_scat(1)
    w_scat(0)

    plsc.subcore_barrier()
    pltpu.sync_copy(agg_sh.at[estripe], out_hbm.at[cid, estripe])
    plsc.subcore_barrier()

    pltpu.sync_copy(zeros_hbm, agg_sh.at[zstripe])
    plsc.subcore_barrier()

    def b_step(j, b, nb, first, do_next):
        w_msg_rd(j, b)
        clamp(j, b, HALF)
        if do_next:
            if not first:
                w_scat(nb)
            i_msg_rd(j + 1, nb)
        i_scat(b)

    i_msg_rd(0, 0)
    b_step(0, 0, 1, True, True)
    b_step(1, 1, 0, False, True)

    def b_pair(jj, carry):
        j = 2 + 2 * jj
        b_step(j, 0, 1, False, True)
        b_step(j + 1, 1, 0, False, True)
        return carry

    lax.fori_loop(0, 61, b_pair, 0)
    b_step(124, 0, 1, False, False)
    w_scat(1)
    w_scat(0)

    plsc.subcore_barrier()
    pltpu.sync_copy(agg_sh.at[estripe],
                    out_hbm.at[cid, pl.ds(HALF + sid * RPE, RPE)])


def kernel(x, edge_index, edge_attr, Wm_d, bm_d, Wu_d, bu_d,
           Wm_r, bm_r, Wu_r, bu_r):
    src = edge_index[0].astype(jnp.int32)
    dst = edge_index[1].astype(jnp.int32)
    ea_flat = edge_attr.reshape(N_EDGES * 4)
    zeros = jnp.zeros((RPZ, D), jnp.float32)

    def layer(xin, s_idx, d_idx, Wm, bm, Wu, bu):
        p = _node_proj(xin, Wm[:D], bm.reshape(1, D))
        parts, _ = _sc_edge(p, ea_flat, Wm[D:], s_idx, d_idx, zeros)
        return _update(xin, parts, Wu[:D], Wu[D:], bu.reshape(1, D))

    h = layer(x, src, dst, Wm_d, bm_d, Wu_d, bu_d)
    h = layer(h, dst, src, Wm_r, bm_r, Wu_r, bu_r)
    return h


# speedup vs baseline: 3.9784x; 3.9784x over previous
import functools

import jax
import jax.numpy as jnp
from jax import lax
from jax.experimental import pallas as pl
from jax.experimental.pallas import tpu as pltpu
from jax.experimental.pallas import tpu_sc as plsc

N_NODES = 10000
N_EDGES = 320000
D = 128

NW = 32
EPW = N_EDGES // NW
C = 80
K = EPW // C
HALF = 5120
AGG_R = 5248
DUMP = HALF
RPE = HALF // 16
RPZ = AGG_R // 16
N_PAD = 10240


def _node_proj_body(x_ref, w_ref, b_ref, o_ref):
    o_ref[...] = jnp.dot(x_ref[...], w_ref[...],
                         preferred_element_type=jnp.float32) + b_ref[...]


def _node_proj(x, w, b):
    br = 2000
    return pl.pallas_call(
        _node_proj_body,
        grid=(N_NODES // br,),
        in_specs=[
            pl.BlockSpec((br, D), lambda i: (i, 0)),
            pl.BlockSpec((D, D), lambda i: (0, 0)),
            pl.BlockSpec((1, D), lambda i: (0, 0)),
        ],
        out_specs=pl.BlockSpec((br, D), lambda i: (i, 0)),
        out_shape=jax.ShapeDtypeStruct((N_NODES, D), jnp.float32),
    )(x, w, b)


def _edge_proj_body(e_ref, w_ref, o_ref):
    e = e_ref[...]
    w = w_ref[...]
    acc = e[:, 0:1] * w[0:1, :]
    for k in range(1, 4):
        acc = acc + e[:, k:k + 1] * w[k:k + 1, :]
    o_ref[...] = acc


def _edge_proj(e, w):
    br = 8000
    return pl.pallas_call(
        _edge_proj_body,
        grid=(N_EDGES // br,),
        in_specs=[
            pl.BlockSpec((br, 4), lambda i: (i, 0)),
            pl.BlockSpec((4, D), lambda i: (0, 0)),
        ],
        out_specs=pl.BlockSpec((br, D), lambda i: (i, 0)),
        out_shape=jax.ShapeDtypeStruct((N_EDGES, D), jnp.float32),
    )(e, w)


def _update_body(x_ref, p_ref, w1_ref, w2_ref, b_ref, o_ref):
    agg = p_ref[0] + p_ref[1]
    o_ref[...] = jnp.tanh(
        jnp.dot(x_ref[...], w1_ref[...], preferred_element_type=jnp.float32)
        + jnp.dot(agg, w2_ref[...], preferred_element_type=jnp.float32)
        + b_ref[...])


def _update(x, parts, w1, w2, b):
    br = 2000
    return pl.pallas_call(
        _update_body,
        grid=(N_NODES // br,),
        in_specs=[
            pl.BlockSpec((br, D), lambda i: (i, 0)),
            pl.BlockSpec((2, br, D), lambda i: (0, i, 0)),
            pl.BlockSpec((D, D), lambda i: (0, 0)),
            pl.BlockSpec((D, D), lambda i: (0, 0)),
            pl.BlockSpec((1, D), lambda i: (0, 0)),
        ],
        out_specs=pl.BlockSpec((br, D), lambda i: (i, 0)),
        out_shape=jax.ShapeDtypeStruct((N_NODES, D), jnp.float32),
    )(x, parts, w1, w2, b)


def _update_fused_body(x_ref, p_ref, w1_ref, w2_ref, b_ref,
                       wn_ref, bn_ref, o_ref, pn_ref):
    agg = p_ref[0] + p_ref[1]
    h = jnp.tanh(
        jnp.dot(x_ref[...], w1_ref[...], preferred_element_type=jnp.float32)
        + jnp.dot(agg, w2_ref[...], preferred_element_type=jnp.float32)
        + b_ref[...])
    o_ref[...] = h
    pn_ref[...] = jnp.dot(h, wn_ref[...],
                          preferred_element_type=jnp.float32) + bn_ref[...]


def _update_fused(x, parts, w1, w2, b, wn, bn):
    br = 2000
    return pl.pallas_call(
        _update_fused_body,
        grid=(N_NODES // br,),
        in_specs=[
            pl.BlockSpec((br, D), lambda i: (i, 0)),
            pl.BlockSpec((2, br, D), lambda i: (0, i, 0)),
            pl.BlockSpec((D, D), lambda i: (0, 0)),
            pl.BlockSpec((D, D), lambda i: (0, 0)),
            pl.BlockSpec((1, D), lambda i: (0, 0)),
            pl.BlockSpec((D, D), lambda i: (0, 0)),
            pl.BlockSpec((1, D), lambda i: (0, 0)),
        ],
        out_specs=[
            pl.BlockSpec((br, D), lambda i: (i, 0)),
            pl.BlockSpec((br, D), lambda i: (i, 0)),
        ],
        out_shape=[
            jax.ShapeDtypeStruct((N_NODES, D), jnp.float32),
            jax.ShapeDtypeStruct((N_NODES, D), jnp.float32),
        ],
    )(x, parts, w1, w2, b, wn, bn)


@functools.partial(
    pl.kernel,
    out_type=[
        jax.ShapeDtypeStruct((2, N_PAD, D), jnp.float32),
        jax.ShapeDtypeStruct((N_EDGES, D), jnp.float32),
    ],
    mesh=plsc.VectorSubcoreMesh(core_axis_name="c", subcore_axis_name="s"),
    scratch_types=[
        pltpu.VMEM((EPW,), jnp.int32),
        pltpu.VMEM((EPW,), jnp.int32),
        pltpu.VMEM((2, C), jnp.int32),
        pltpu.VMEM((2, C, D), jnp.float32),
        pltpu.VMEM((2, C, D), jnp.float32),
        pltpu.VMEM_SHARED((AGG_R, D), jnp.float32),
    ] + [pltpu.SemaphoreType.DMA] * 8,
)
def _sc_edge(p_hbm, ep_hbm, src_hbm, dst_hbm, zeros_hbm,
             out_hbm, msg_hbm,
             src_sl, dst_sl, idxt_b, rows_b, ep_b, agg_sh,
             se0, se1, sg0, sg1, sp0, sp1, sc0, sc1):
    s_ep = (se0, se1)
    s_g = (sg0, sg1)
    s_sp = (sp0, sp1)
    s_sc = (sc0, sc1)

    cid = lax.axis_index("c")
    sid = lax.axis_index("s")
    wid = sid * 2 + cid
    base0 = wid * EPW
    zstripe = pl.ds(sid * RPZ, RPZ)
    estripe = pl.ds(sid * RPE, RPE)

    def esl(j):
        return pl.ds(base0 + j * C, C)

    def i_ep(j, b):
        pltpu.async_copy(ep_hbm.at[esl(j)], ep_b.at[b], s_ep[b])

    def w_ep(j, b):
        pltpu.make_async_copy(ep_hbm.at[esl(j)], ep_b.at[b], s_ep[b]).wait()

    def i_gather(j, b):
        pltpu.async_copy(p_hbm.at[src_sl.at[pl.ds(j * C, C)]],
                         rows_b.at[b], s_g[b])

    def w_gather(j, b):
        pltpu.make_async_copy(p_hbm.at[src_sl.at[pl.ds(j * C, C)]],
                              rows_b.at[b], s_g[b]).wait()

    def i_msg_rd(j, b):
        pltpu.async_copy(msg_hbm.at[esl(j)], rows_b.at[b], s_g[b])

    def w_msg_rd(j, b):
        pltpu.make_async_copy(msg_hbm.at[esl(j)], rows_b.at[b], s_g[b]).wait()

    def i_spill(j, b):
        pltpu.async_copy(rows_b.at[b], msg_hbm.at[esl(j)], s_sp[b])

    def w_spill(j, b):
        pltpu.make_async_copy(rows_b.at[b], msg_hbm.at[esl(j)],
                              s_sp[b]).wait()

    def i_scat(b):
        pltpu.async_copy(rows_b.at[b], agg_sh.at[idxt_b.at[b]], s_sc[b],
                         add=True)

    def w_scat(b):
        pltpu.make_async_copy(rows_b.at[b], agg_sh.at[idxt_b.at[b]],
                              s_sc[b]).wait()

    def compute(b):
        def erow(i, c2):
            for k8 in range(D // 16):
                sl = pl.ds(k8 * 16, 16)
                g = rows_b[b, i, sl] + ep_b[b, i, sl]
                u = jnp.exp(jnp.clip(2.0 * g, -30.0, 30.0))
                rows_b[b, i, sl] = 1.0 - 2.0 / (u + 1.0)
            return c2

        lax.fori_loop(0, C, erow, 0)

    def clamp(j, b, lo):
        for k in range(C // 16):
            v = dst_sl[pl.ds(j * C + k * 16, 16)] - lo
            ok = jnp.logical_and(v >= 0, v < HALF)
            dump = DUMP + k * 16 + lax.broadcasted_iota(jnp.int32, (16,), 0)
            idxt_b[b, pl.ds(k * 16, 16)] = jnp.where(ok, v, dump)

    pltpu.sync_copy(src_hbm.at[pl.ds(base0, EPW)], src_sl)
    pltpu.sync_copy(dst_hbm.at[pl.ds(base0, EPW)], dst_sl)
    pltpu.sync_copy(zeros_hbm, agg_sh.at[zstripe])
    plsc.subcore_barrier()


    def a_step(j, b, nb, first, do_next_gather):
        if do_next_gather:
            if not first:
                w_spill(j - 1, nb)
                w_scat(nb)
            i_gather(j + 1, nb)
        w_gather(j, b)
        w_ep(j, b)
        compute(b)
        clamp(j, b, 0)
        i_spill(j, b)
        i_scat(b)

    i_ep(0, 0)
    i_ep(1, 1)
    i_gather(0, 0)
    a_step(0, 0, 1, True, True)
    i_ep(2, 0)

    def a_pair(jj, carry):
        j = 1 + 2 * jj
        a_step(j, 1, 0, False, True)
        i_ep(j + 2, 1)
        a_step(j + 1, 0, 1, False, True)
        i_ep(j + 3, 0)
        return carry

    lax.fori_loop(0, 61, a_pair, 0)
    a_step(123, 1, 0, False, True)
    a_step(124, 0, 1, False, False)
    w_spill(123, 1)
    w_spill(124, 0)
    w_scat(1)
    w_scat(0)

    plsc.subcore_barrier()
    pltpu.sync_copy(agg_sh.at[estripe], out_hbm.at[cid, estripe])
    plsc.subcore_barrier()

    pltpu.sync_copy(zeros_hbm, agg_sh.at[zstripe])
    plsc.subcore_barrier()

    def b_step(j, b, nb, first, do_next):
        w_msg_rd(j, b)
        clamp(j, b, HALF)
        if do_next:
            if not first:
                w_scat(nb)
            i_msg_rd(j + 1, nb)
        i_scat(b)

    i_msg_rd(0, 0)
    b_step(0, 0, 1, True, True)
    b_step(1, 1, 0, False, True)

    def b_pair(jj, carry):
        j = 2 + 2 * jj
        b_step(j, 0, 1, False, True)
        b_step(j + 1, 1, 0, False, True)
        return carry

    lax.fori_loop(0, 61, b_pair, 0)
    b_step(124, 0, 1, False, False)
    w_scat(1)
    w_scat(0)

    plsc.subcore_barrier()
    pltpu.sync_copy(agg_sh.at[estripe],
                    out_hbm.at[cid, pl.ds(HALF + sid * RPE, RPE)])


def kernel(x, edge_index, edge_attr, Wm_d, bm_d, Wu_d, bu_d,
           Wm_r, bm_r, Wu_r, bu_r):
    src = edge_index[0].astype(jnp.int32)
    dst = edge_index[1].astype(jnp.int32)
    zeros = jnp.zeros((RPZ, D), jnp.float32)

    p1 = _node_proj(x, Wm_d[:D], bm_d.reshape(1, D))
    ep1 = _edge_proj(edge_attr, Wm_d[D:])
    parts1, _ = _sc_edge(p1, ep1, src, dst, zeros)
    h1, p2 = _update_fused(x, parts1, Wu_d[:D], Wu_d[D:],
                           bu_d.reshape(1, D), Wm_r[:D], bm_r.reshape(1, D))
    ep2 = _edge_proj(edge_attr, Wm_r[D:])
    parts2, _ = _sc_edge(p2, ep2, dst, src, zeros)
    return _update(h1, parts2, Wu_r[:D], Wu_r[D:], bu_r.reshape(1, D))


# speedup vs baseline: 4.7354x; 1.1903x over previous
import functools

import jax
import jax.numpy as jnp
from jax import lax
from jax.experimental import pallas as pl
from jax.experimental.pallas import tpu as pltpu
from jax.experimental.pallas import tpu_sc as plsc

N_NODES = 10000
N_EDGES = 320000
D = 128

NW = 32
EPW = N_EDGES // NW
C = 80
K = EPW // C
HALF = 5120
AGG_R = 5248
DUMP = HALF
RPE = HALF // 16
RPZ = AGG_R // 16
N_PAD = 10240


def _node_proj_body(x_ref, w_ref, b_ref, o_ref):
    o_ref[...] = jnp.dot(x_ref[...], w_ref[...],
                         preferred_element_type=jnp.float32) + b_ref[...]


def _node_proj(x, w, b):
    br = 2000
    return pl.pallas_call(
        _node_proj_body,
        grid=(N_NODES // br,),
        in_specs=[
            pl.BlockSpec((br, D), lambda i: (i, 0)),
            pl.BlockSpec((D, D), lambda i: (0, 0)),
            pl.BlockSpec((1, D), lambda i: (0, 0)),
        ],
        out_specs=pl.BlockSpec((br, D), lambda i: (i, 0)),
        out_shape=jax.ShapeDtypeStruct((N_NODES, D), jnp.float32),
    )(x, w, b)


def _edge_proj_body(e_ref, w_ref, o_ref):
    e = e_ref[...]
    w = w_ref[...]
    acc = e[:, 0:1] * w[0:1, :]
    for k in range(1, 4):
        acc = acc + e[:, k:k + 1] * w[k:k + 1, :]
    o_ref[...] = acc


def _edge_proj(e, w):
    br = 8000
    return pl.pallas_call(
        _edge_proj_body,
        grid=(N_EDGES // br,),
        in_specs=[
            pl.BlockSpec((br, 4), lambda i: (i, 0)),
            pl.BlockSpec((4, D), lambda i: (0, 0)),
        ],
        out_specs=pl.BlockSpec((br, D), lambda i: (i, 0)),
        out_shape=jax.ShapeDtypeStruct((N_EDGES, D), jnp.float32),
    )(e, w)


def _update_body(x_ref, p_ref, w1_ref, w2_ref, b_ref, o_ref):
    agg = p_ref[0] + p_ref[1]
    o_ref[...] = jnp.tanh(
        jnp.dot(x_ref[...], w1_ref[...], preferred_element_type=jnp.float32)
        + jnp.dot(agg, w2_ref[...], preferred_element_type=jnp.float32)
        + b_ref[...])


def _update(x, parts, w1, w2, b):
    br = 2000
    return pl.pallas_call(
        _update_body,
        grid=(N_NODES // br,),
        in_specs=[
            pl.BlockSpec((br, D), lambda i: (i, 0)),
            pl.BlockSpec((2, br, D), lambda i: (0, i, 0)),
            pl.BlockSpec((D, D), lambda i: (0, 0)),
            pl.BlockSpec((D, D), lambda i: (0, 0)),
            pl.BlockSpec((1, D), lambda i: (0, 0)),
        ],
        out_specs=pl.BlockSpec((br, D), lambda i: (i, 0)),
        out_shape=jax.ShapeDtypeStruct((N_NODES, D), jnp.float32),
    )(x, parts, w1, w2, b)


def _update_fused_body(x_ref, p_ref, w1_ref, w2_ref, b_ref,
                       wn_ref, bn_ref, o_ref, pn_ref):
    agg = p_ref[0] + p_ref[1]
    h = jnp.tanh(
        jnp.dot(x_ref[...], w1_ref[...], preferred_element_type=jnp.float32)
        + jnp.dot(agg, w2_ref[...], preferred_element_type=jnp.float32)
        + b_ref[...])
    o_ref[...] = h
    pn_ref[...] = jnp.dot(h, wn_ref[...],
                          preferred_element_type=jnp.float32) + bn_ref[...]


def _update_fused(x, parts, w1, w2, b, wn, bn):
    br = 2000
    return pl.pallas_call(
        _update_fused_body,
        grid=(N_NODES // br,),
        in_specs=[
            pl.BlockSpec((br, D), lambda i: (i, 0)),
            pl.BlockSpec((2, br, D), lambda i: (0, i, 0)),
            pl.BlockSpec((D, D), lambda i: (0, 0)),
            pl.BlockSpec((D, D), lambda i: (0, 0)),
            pl.BlockSpec((1, D), lambda i: (0, 0)),
            pl.BlockSpec((D, D), lambda i: (0, 0)),
            pl.BlockSpec((1, D), lambda i: (0, 0)),
        ],
        out_specs=[
            pl.BlockSpec((br, D), lambda i: (i, 0)),
            pl.BlockSpec((br, D), lambda i: (i, 0)),
        ],
        out_shape=[
            jax.ShapeDtypeStruct((N_NODES, D), jnp.float32),
            jax.ShapeDtypeStruct((N_NODES, D), jnp.float32),
        ],
    )(x, parts, w1, w2, b, wn, bn)


@functools.partial(
    pl.kernel,
    out_type=[
        jax.ShapeDtypeStruct((2, N_PAD, D), jnp.float32),
        jax.ShapeDtypeStruct((N_EDGES, D), jnp.float32),
    ],
    mesh=plsc.VectorSubcoreMesh(core_axis_name="c", subcore_axis_name="s"),
    scratch_types=[
        pltpu.VMEM((EPW,), jnp.int32),
        pltpu.VMEM((EPW,), jnp.int32),
        pltpu.VMEM((2, C), jnp.int32),
        pltpu.VMEM((2, C, D), jnp.float32),
        pltpu.VMEM((2, C, D), jnp.float32),
        pltpu.VMEM_SHARED((AGG_R, D), jnp.float32),
    ] + [pltpu.SemaphoreType.DMA] * 8,
)
def _sc_edge(p_hbm, ep_hbm, src_hbm, dst_hbm, zeros_hbm,
             out_hbm, msg_hbm,
             src_sl, dst_sl, idxt_b, rows_b, ep_b, agg_sh,
             se0, se1, sg0, sg1, sp0, sp1, sc0, sc1):
    s_ep = (se0, se1)
    s_g = (sg0, sg1)
    s_sp = (sp0, sp1)
    s_sc = (sc0, sc1)

    cid = lax.axis_index("c")
    sid = lax.axis_index("s")
    wid = sid * 2 + cid
    base0 = wid * EPW
    zstripe = pl.ds(sid * RPZ, RPZ)
    estripe = pl.ds(sid * RPE, RPE)

    def esl(j):
        return pl.ds(base0 + j * C, C)

    def i_ep(j, b):
        pltpu.async_copy(ep_hbm.at[esl(j)], ep_b.at[b], s_ep[b])

    def w_ep(j, b):
        pltpu.make_async_copy(ep_hbm.at[esl(j)], ep_b.at[b], s_ep[b]).wait()

    def i_gather(j, b):
        pltpu.async_copy(p_hbm.at[src_sl.at[pl.ds(j * C, C)]],
                         rows_b.at[b], s_g[b])

    def w_gather(j, b):
        pltpu.make_async_copy(p_hbm.at[src_sl.at[pl.ds(j * C, C)]],
                              rows_b.at[b], s_g[b]).wait()

    def i_msg_rd(j, b):
        pltpu.async_copy(msg_hbm.at[esl(j)], rows_b.at[b], s_g[b])

    def w_msg_rd(j, b):
        pltpu.make_async_copy(msg_hbm.at[esl(j)], rows_b.at[b], s_g[b]).wait()

    def i_spill(j, b):
        pltpu.async_copy(rows_b.at[b], msg_hbm.at[esl(j)], s_sp[b])

    def w_spill(j, b):
        pltpu.make_async_copy(rows_b.at[b], msg_hbm.at[esl(j)],
                              s_sp[b]).wait()

    def i_scat(b):
        pltpu.async_copy(rows_b.at[b], agg_sh.at[idxt_b.at[b]], s_sc[b],
                         add=True)

    def w_scat(b):
        pltpu.make_async_copy(rows_b.at[b], agg_sh.at[idxt_b.at[b]],
                              s_sc[b]).wait()

    def compute(b):
        @functools.partial(plsc.parallel_loop, 0, C, unroll=2)
        def erow(i):
            for k8 in range(D // 16):
                sl = pl.ds(k8 * 16, 16)
                g = rows_b[b, i, sl] + ep_b[b, i, sl]
                u = jnp.exp(jnp.clip(2.0 * g, -30.0, 30.0))
                rows_b[b, i, sl] = 1.0 - 2.0 / (u + 1.0)

    def clamp(j, b, lo):
        for k in range(C // 16):
            v = dst_sl[pl.ds(j * C + k * 16, 16)] - lo
            ok = jnp.logical_and(v >= 0, v < HALF)
            dump = DUMP + k * 16 + lax.broadcasted_iota(jnp.int32, (16,), 0)
            idxt_b[b, pl.ds(k * 16, 16)] = jnp.where(ok, v, dump)

    pltpu.sync_copy(src_hbm.at[pl.ds(base0, EPW)], src_sl)
    pltpu.sync_copy(dst_hbm.at[pl.ds(base0, EPW)], dst_sl)
    pltpu.sync_copy(zeros_hbm, agg_sh.at[zstripe])
    plsc.subcore_barrier()


    def a_step(j, b, nb, first, do_next_gather):
        if do_next_gather:
            if not first:
                w_spill(j - 1, nb)
                w_scat(nb)
            i_gather(j + 1, nb)
        w_gather(j, b)
        w_ep(j, b)
        compute(b)
        clamp(j, b, 0)
        i_spill(j, b)
        i_scat(b)

    i_ep(0, 0)
    i_ep(1, 1)
    i_gather(0, 0)
    a_step(0, 0, 1, True, True)
    i_ep(2, 0)

    def a_pair(jj, carry):
        j = 1 + 2 * jj
        a_step(j, 1, 0, False, True)
        i_ep(j + 2, 1)
        a_step(j + 1, 0, 1, False, True)
        i_ep(j + 3, 0)
        return carry

    lax.fori_loop(0, 61, a_pair, 0)
    a_step(123, 1, 0, False, True)
    a_step(124, 0, 1, False, False)
    w_spill(123, 1)
    w_spill(124, 0)
    w_scat(1)
    w_scat(0)

    plsc.subcore_barrier()
    pltpu.sync_copy(agg_sh.at[estripe], out_hbm.at[cid, estripe])
    plsc.subcore_barrier()

    pltpu.sync_copy(zeros_hbm, agg_sh.at[zstripe])
    plsc.subcore_barrier()

    def b_step(j, b, nb, first, do_next):
        w_msg_rd(j, b)
        clamp(j, b, HALF)
        if do_next:
            if not first:
                w_scat(nb)
            i_msg_rd(j + 1, nb)
        i_scat(b)

    i_msg_rd(0, 0)
    b_step(0, 0, 1, True, True)
    b_step(1, 1, 0, False, True)

    def b_pair(jj, carry):
        j = 2 + 2 * jj
        b_step(j, 0, 1, False, True)
        b_step(j + 1, 1, 0, False, True)
        return carry

    lax.fori_loop(0, 61, b_pair, 0)
    b_step(124, 0, 1, False, False)
    w_scat(1)
    w_scat(0)

    plsc.subcore_barrier()
    pltpu.sync_copy(agg_sh.at[estripe],
                    out_hbm.at[cid, pl.ds(HALF + sid * RPE, RPE)])


def kernel(x, edge_index, edge_attr, Wm_d, bm_d, Wu_d, bu_d,
           Wm_r, bm_r, Wu_r, bu_r):
    src = edge_index[0].astype(jnp.int32)
    dst = edge_index[1].astype(jnp.int32)
    zeros = jnp.zeros((RPZ, D), jnp.float32)

    p1 = _node_proj(x, Wm_d[:D], bm_d.reshape(1, D))
    ep1 = _edge_proj(edge_attr, Wm_d[D:])
    parts1, _ = _sc_edge(p1, ep1, src, dst, zeros)
    h1, p2 = _update_fused(x, parts1, Wu_d[:D], Wu_d[D:],
                           bu_d.reshape(1, D), Wm_r[:D], bm_r.reshape(1, D))
    ep2 = _edge_proj(edge_attr, Wm_r[D:])
    parts2, _ = _sc_edge(p2, ep2, dst, src, zeros)
    return _update(h1, parts2, Wu_r[:D], Wu_r[D:], bu_r.reshape(1, D))
